# baseline (device time: 29636 ns/iter reference)
import jax
import jax.numpy as jnp
from jax import lax
from jax.experimental import pallas as pl
from jax.experimental.pallas import tpu as pltpu

N_DEV = 8
N_ROWS = 1024
D_MODEL = 256
N_EXPERTS = 32
E_LOCAL = 4
H = 512
ROWS_PER = N_ROWS // N_DEV


def kernel(x, router_W, route_idx, expert_W, shared_W):
    def body(x_ref, rw_ref, idx_ref, ew_ref, sw_ref, out_ref,
             partial_ref, comm_ref, send_sems, recv_sems):
        my = lax.axis_index("i")

        barrier_sem = pltpu.get_barrier_semaphore()
        for k in range(1, N_DEV):
            pl.semaphore_signal(
                barrier_sem, inc=1,
                device_id=((my + k) % N_DEV,),
                device_id_type=pl.DeviceIdType.MESH,
            )
        pl.semaphore_wait(barrier_sem, N_DEV - 1)

        xv = x_ref[:, :]
        scores = jnp.dot(xv, rw_ref[:, :], preferred_element_type=jnp.float32)
        m = jnp.max(scores, axis=-1, keepdims=True)
        e = jnp.exp(scores - m)
        probs = e / jnp.sum(e, axis=-1, keepdims=True)
        idx = idx_ref[:, :]
        onehot = (idx == lax.broadcasted_iota(jnp.int32, (N_ROWS, N_EXPERTS), 1))
        gate = jnp.sum(probs * onehot.astype(jnp.float32), axis=-1,
                       keepdims=True)
        local_ids = my * E_LOCAL + lax.broadcasted_iota(
            jnp.int32, (N_ROWS, E_LOCAL), 1)
        w = gate * (idx == local_ids).astype(jnp.float32)

        acc = jnp.zeros((N_ROWS, H), jnp.float32)
        for j in range(E_LOCAL):
            acc = acc + jnp.dot(w[:, j:j + 1] * xv, ew_ref[j],
                                preferred_element_type=jnp.float32)
        partial_ref[:, :] = acc

        rdmas = []
        for k in range(1, N_DEV):
            d = (my + k) % N_DEV
            rdma = pltpu.make_async_remote_copy(
                src_ref=partial_ref.at[pl.ds(d * ROWS_PER, ROWS_PER), :],
                dst_ref=comm_ref.at[k - 1],
                send_sem=send_sems.at[k - 1],
                recv_sem=recv_sems.at[k - 1],
                device_id=(d,),
                device_id_type=pl.DeviceIdType.MESH,
            )
            rdma.start()
            rdmas.append(rdma)

        x_my = x_ref[pl.ds(my * ROWS_PER, ROWS_PER), :]
        total = jnp.dot(x_my, sw_ref[:, :], preferred_element_type=jnp.float32)
        total = total + partial_ref[pl.ds(my * ROWS_PER, ROWS_PER), :]
        for k in range(1, N_DEV):
            rdmas[k - 1].wait_recv()
            total = total + comm_ref[k - 1]
        out_ref[:, :] = total
        for k in range(1, N_DEV):
            rdmas[k - 1].wait_send()

    return pl.pallas_call(
        body,
        out_shape=jax.ShapeDtypeStruct((ROWS_PER, H), jnp.float32),
        in_specs=[pl.BlockSpec(memory_space=pltpu.VMEM)] * 5,
        out_specs=pl.BlockSpec(memory_space=pltpu.VMEM),
        scratch_shapes=[
            pltpu.VMEM((N_ROWS, H), jnp.float32),
            pltpu.VMEM((N_DEV - 1, ROWS_PER, H), jnp.float32),
            pltpu.SemaphoreType.DMA((N_DEV - 1,)),
            pltpu.SemaphoreType.DMA((N_DEV - 1,)),
        ],
        compiler_params=pltpu.CompilerParams(collective_id=0),
    )(x, router_W, route_idx, expert_W, shared_W)


# device time: 29050 ns/iter; 1.0202x vs baseline; 1.0202x over previous
import jax
import jax.numpy as jnp
from jax import lax
from jax.experimental import pallas as pl
from jax.experimental.pallas import tpu as pltpu

N_DEV = 8
N_ROWS = 1024
D_MODEL = 256
N_EXPERTS = 32
E_LOCAL = 4
H = 512
ROWS_PER = N_ROWS // N_DEV


def kernel(x, router_W, route_idx, expert_W, shared_W):
    def body(x_ref, rw_ref, idx_ref, ew_ref, sw_ref, out_ref,
             partial_ref, comm_ref, xw_ref, send_sems, recv_sems):
        my = lax.axis_index("i")

        barrier_sem = pltpu.get_barrier_semaphore()
        for k in range(1, N_DEV):
            pl.semaphore_signal(
                barrier_sem, inc=1,
                device_id=((my + k) % N_DEV,),
                device_id_type=pl.DeviceIdType.MESH,
            )
        pl.semaphore_wait(barrier_sem, N_DEV - 1)

        xv = x_ref[:, :]
        scores = jnp.dot(xv, rw_ref[:, :], preferred_element_type=jnp.float32)
        m = jnp.max(scores, axis=-1, keepdims=True)
        e = jnp.exp(scores - m)
        probs = e / jnp.sum(e, axis=-1, keepdims=True)
        idx = idx_ref[:, :]
        onehot = (idx == lax.broadcasted_iota(jnp.int32, (N_ROWS, N_EXPERTS), 1))
        gate = jnp.sum(probs * onehot.astype(jnp.float32), axis=-1,
                       keepdims=True)
        local_ids = my * E_LOCAL + lax.broadcasted_iota(
            jnp.int32, (N_ROWS, E_LOCAL), 1)
        w = gate * (idx == local_ids).astype(jnp.float32)

        for j in range(E_LOCAL):
            xw_ref[j] = w[:, j:j + 1] * xv

        rdmas = []
        for k in range(1, N_DEV):
            d = (my + k) % N_DEV
            rows = pl.ds(d * ROWS_PER, ROWS_PER)
            blk = jnp.zeros((ROWS_PER, H), jnp.float32)
            for j in range(E_LOCAL):
                blk = blk + jnp.dot(xw_ref[j, rows, :], ew_ref[j],
                                    preferred_element_type=jnp.float32)
            partial_ref[rows, :] = blk
            rdma = pltpu.make_async_remote_copy(
                src_ref=partial_ref.at[rows, :],
                dst_ref=comm_ref.at[k - 1],
                send_sem=send_sems.at[k - 1],
                recv_sem=recv_sems.at[k - 1],
                device_id=(d,),
                device_id_type=pl.DeviceIdType.MESH,
            )
            rdma.start()
            rdmas.append(rdma)

        my_rows = pl.ds(my * ROWS_PER, ROWS_PER)
        x_my = x_ref[my_rows, :]
        total = jnp.dot(x_my, sw_ref[:, :], preferred_element_type=jnp.float32)
        for j in range(E_LOCAL):
            total = total + jnp.dot(xw_ref[j, my_rows, :], ew_ref[j],
                                    preferred_element_type=jnp.float32)
        for k in range(1, N_DEV):
            rdmas[k - 1].wait_recv()
            total = total + comm_ref[k - 1]
        out_ref[:, :] = total
        for k in range(1, N_DEV):
            rdmas[k - 1].wait_send()

    return pl.pallas_call(
        body,
        out_shape=jax.ShapeDtypeStruct((ROWS_PER, H), jnp.float32),
        in_specs=[pl.BlockSpec(memory_space=pltpu.VMEM)] * 5,
        out_specs=pl.BlockSpec(memory_space=pltpu.VMEM),
        scratch_shapes=[
            pltpu.VMEM((N_ROWS, H), jnp.float32),
            pltpu.VMEM((N_DEV - 1, ROWS_PER, H), jnp.float32),
            pltpu.VMEM((E_LOCAL, N_ROWS, D_MODEL), jnp.float32),
            pltpu.SemaphoreType.DMA((N_DEV - 1,)),
            pltpu.SemaphoreType.DMA((N_DEV - 1,)),
        ],
        compiler_params=pltpu.CompilerParams(collective_id=0),
    )(x, router_W, route_idx, expert_W, shared_W)
